# baseline (speedup 1.0000x reference)
import functools

import jax
import jax.numpy as jnp
from jax.experimental import pallas as pl
from jax.experimental.pallas import tpu as pltpu

N_E = 1024
E_DIM = 64
BETA = 0.25
TN = 1024


def _vq_kernel(z3_ref, wt_ref, oh_ref, zq3_ref, idx_ref, loss_ref,
               counts_ref, perp_ref, *, n_total, n_steps):
    step = pl.program_id(0)

    zt = z3_ref[0]
    z2t = zt + zt
    z2 = jnp.transpose(z2t)
    wt = wt_ref[...]

    dot2 = jax.lax.dot_general(z2, wt, (((1,), (0,)), ((), ())),
                               preferred_element_type=jnp.float32)
    z_sq = 0.25 * jnp.sum(z2 * z2, axis=1, keepdims=True)
    e_sq = jnp.sum(wt * wt, axis=0, keepdims=True)
    d = (z_sq + e_sq) - dot2

    d_min = jnp.min(d, axis=1, keepdims=True)
    fiota = jax.lax.broadcasted_iota(jnp.int32, (TN, N_E), 1).astype(jnp.float32)
    idx_f = jnp.min(jnp.where(d == d_min, fiota, float(N_E)),
                    axis=1, keepdims=True)
    idx_ref[...] = idx_f.astype(jnp.int32)

    one_hot = (fiota == idx_f).astype(jnp.float32)
    oh_ref[...] = one_hot

    zq = jax.lax.dot_general(one_hot, jnp.transpose(wt),
                             (((1,), (0,)), ((), ())),
                             preferred_element_type=jnp.float32)
    zqt = jnp.transpose(zq)
    zq3_ref[0] = zqt

    @pl.when(step == 0)
    def _init():
        loss_ref[...] = jnp.zeros_like(loss_ref)
        counts_ref[...] = jnp.zeros_like(counts_ref)
        perp_ref[...] = jnp.zeros_like(perp_ref)

    diff = zqt - 0.5 * z2t
    sq = jnp.sum(diff * diff)
    loss_ref[...] += jnp.full(loss_ref.shape, sq, jnp.float32)
    counts_ref[...] += jnp.sum(one_hot, axis=0, keepdims=True)

    @pl.when(step == n_steps - 1)
    def _finalize():
        loss_ref[...] = loss_ref[...] * (BETA / (n_total * E_DIM))
        p = counts_ref[...] / n_total
        ent = -jnp.sum(p * jnp.log(p + 1e-10))
        perp_ref[...] = jnp.full(perp_ref.shape, jnp.exp(ent), jnp.float32)


def kernel(z, W):
    B, C, H, Wd = z.shape
    n = B * H * Wd
    n_steps = n // TN
    z3 = z.reshape(B, C, H * Wd)
    wt = W.T

    grid = (n_steps,)
    out_shapes = (
        jax.ShapeDtypeStruct((n, N_E), jnp.float32),
        jax.ShapeDtypeStruct((B, C, H * Wd), jnp.float32),
        jax.ShapeDtypeStruct((n, 1), jnp.int32),
        jax.ShapeDtypeStruct((1, 128), jnp.float32),
        jax.ShapeDtypeStruct((1, N_E), jnp.float32),
        jax.ShapeDtypeStruct((1, 128), jnp.float32),
    )
    in_specs = [
        pl.BlockSpec((1, C, H * Wd), lambda i: (i, 0, 0)),
        pl.BlockSpec((E_DIM, N_E), lambda i: (0, 0)),
    ]
    out_specs = (
        pl.BlockSpec((TN, N_E), lambda i: (i, 0)),
        pl.BlockSpec((1, C, H * Wd), lambda i: (i, 0, 0)),
        pl.BlockSpec((TN, 1), lambda i: (i, 0)),
        pl.BlockSpec((1, 128), lambda i: (0, 0)),
        pl.BlockSpec((1, N_E), lambda i: (0, 0)),
        pl.BlockSpec((1, 128), lambda i: (0, 0)),
    )
    one_hot, zq3, idx_col, loss_o, _counts, perp_o = pl.pallas_call(
        functools.partial(_vq_kernel, n_total=n, n_steps=n_steps),
        grid=grid,
        in_specs=in_specs,
        out_specs=out_specs,
        out_shape=out_shapes,
        compiler_params=pltpu.CompilerParams(
            dimension_semantics=("arbitrary",)),
    )(z3, wt)

    z_q = zq3.reshape(B, C, H, Wd)
    indices = idx_col.reshape(n)
    loss = loss_o[0, 0]
    perplexity = perp_o[0, 0]
    return (loss, z_q, perplexity, one_hot, indices)


# speedup vs baseline: 1.2545x; 1.2545x over previous
import functools

import jax
import jax.numpy as jnp
from jax.experimental import pallas as pl
from jax.experimental.pallas import tpu as pltpu

N_E = 1024
E_DIM = 64
BETA = 0.25
TN = 1024


def _vq_kernel(z_ref, wt_ref, w_ref, oh_ref, zq_ref, idx_ref, loss_ref,
               counts_ref, perp_ref, *, n_total, n_steps):
    step = pl.program_id(0)

    z = z_ref[...]
    z2 = z + z
    wt = wt_ref[...]

    dot2 = jax.lax.dot_general(z2, wt, (((1,), (0,)), ((), ())),
                               preferred_element_type=jnp.float32)
    z_sq = 0.25 * jnp.sum(z2 * z2, axis=1, keepdims=True)
    e_sq = jnp.sum(wt * wt, axis=0, keepdims=True)
    d = (z_sq + e_sq) - dot2

    d_min = jnp.min(d, axis=1, keepdims=True)
    fiota = jax.lax.broadcasted_iota(jnp.int32, (TN, N_E), 1).astype(jnp.float32)
    idx_f = jnp.min(jnp.where(d == d_min, fiota, float(N_E)),
                    axis=1, keepdims=True)
    idx_ref[...] = idx_f.astype(jnp.int32)

    one_hot = (fiota == idx_f).astype(jnp.float32)
    oh_ref[...] = one_hot

    zq = jax.lax.dot_general(one_hot, w_ref[...], (((1,), (0,)), ((), ())),
                             preferred_element_type=jnp.float32)
    zq_ref[...] = zq

    @pl.when(step == 0)
    def _init():
        loss_ref[...] = jnp.zeros_like(loss_ref)
        counts_ref[...] = jnp.zeros_like(counts_ref)
        perp_ref[...] = jnp.zeros_like(perp_ref)

    diff = zq - z
    sq = jnp.sum(diff * diff)
    loss_ref[...] += jnp.full(loss_ref.shape, sq, jnp.float32)
    counts_ref[...] += jnp.sum(one_hot, axis=0, keepdims=True)

    @pl.when(step == n_steps - 1)
    def _finalize():
        loss_ref[...] = loss_ref[...] * (BETA / (n_total * E_DIM))
        p = counts_ref[...] / n_total
        ent = -jnp.sum(p * jnp.log(p + 1e-10))
        perp_ref[...] = jnp.full(perp_ref.shape, jnp.exp(ent), jnp.float32)


def kernel(z, W):
    B, C, H, Wd = z.shape
    n = B * H * Wd
    n_steps = n // TN
    z_flat = jnp.transpose(z, (0, 2, 3, 1)).reshape(n, E_DIM)
    wt = W.T

    grid = (n_steps,)
    out_shapes = (
        jax.ShapeDtypeStruct((n, N_E), jnp.float32),
        jax.ShapeDtypeStruct((n, E_DIM), jnp.float32),
        jax.ShapeDtypeStruct((n, 1), jnp.int32),
        jax.ShapeDtypeStruct((1, 128), jnp.float32),
        jax.ShapeDtypeStruct((1, N_E), jnp.float32),
        jax.ShapeDtypeStruct((1, 128), jnp.float32),
    )
    in_specs = [
        pl.BlockSpec((TN, E_DIM), lambda i: (i, 0)),
        pl.BlockSpec((E_DIM, N_E), lambda i: (0, 0)),
        pl.BlockSpec((N_E, E_DIM), lambda i: (0, 0)),
    ]
    out_specs = (
        pl.BlockSpec((TN, N_E), lambda i: (i, 0)),
        pl.BlockSpec((TN, E_DIM), lambda i: (i, 0)),
        pl.BlockSpec((TN, 1), lambda i: (i, 0)),
        pl.BlockSpec((1, 128), lambda i: (0, 0)),
        pl.BlockSpec((1, N_E), lambda i: (0, 0)),
        pl.BlockSpec((1, 128), lambda i: (0, 0)),
    )
    one_hot, zq_flat, idx_col, loss_o, _counts, perp_o = pl.pallas_call(
        functools.partial(_vq_kernel, n_total=n, n_steps=n_steps),
        grid=grid,
        in_specs=in_specs,
        out_specs=out_specs,
        out_shape=out_shapes,
        compiler_params=pltpu.CompilerParams(
            dimension_semantics=("arbitrary",)),
    )(z_flat, wt, W)

    z_q = jnp.transpose(zq_flat.reshape(B, H, Wd, E_DIM), (0, 3, 1, 2))
    indices = idx_col.reshape(n)
    loss = loss_o[0, 0]
    perplexity = perp_o[0, 0]
    return (loss, z_q, perplexity, one_hot, indices)
